# baseline (speedup 1.0000x reference)
import jax
import jax.numpy as jnp
from jax import lax
from jax.experimental import pallas as pl
from jax.experimental.pallas import tpu as pltpu
from jax.experimental.pallas import tpu_sc as plsc

_B = 16384
_D = 64
_NC = 2
_NS = 16
_NW = _NC * _NS
_BPW = _B // _NW
_CHUNK = 128
_NCHUNK = _BPW // _CHUNK
_L = 16


def _body(users_hbm, items_hbm, utT_hbm, itT_hbm, out_hbm,
          idx_u, idx_i, u_cols, i_cols, out_v, sem_u, sem_i):
    wid = lax.axis_index("s") * _NC + lax.axis_index("c")
    base = wid * _BPW

    for c in range(_NCHUNK):
        off = base + c * _CHUNK
        pltpu.sync_copy(users_hbm.at[pl.ds(off, _CHUNK)], idx_u.at[c])
        pltpu.sync_copy(items_hbm.at[pl.ds(off, _CHUNK)], idx_i.at[c])

    def fire(d, carry):
        for c in range(_NCHUNK):
            pltpu.async_copy(utT_hbm.at[d].at[idx_u.at[c]],
                             u_cols.at[d, pl.ds(c * _CHUNK, _CHUNK)], sem_u)
            pltpu.async_copy(itT_hbm.at[d].at[idx_i.at[c]],
                             i_cols.at[d, pl.ds(c * _CHUNK, _CHUNK)], sem_i)
        return carry

    lax.fori_loop(0, _D, fire, 0)
    pltpu.make_async_copy(utT_hbm.at[pl.ds(0, _D), pl.ds(0, _BPW)], u_cols,
                          sem_u).wait()
    pltpu.make_async_copy(itT_hbm.at[pl.ds(0, _D), pl.ds(0, _BPW)], i_cols,
                          sem_i).wait()

    def group(g, carry):
        def dstep(d0, acc):
            for dd in range(8):
                d = d0 * 8 + dd
                u = u_cols[d, pl.ds(g * _L, _L)]
                v = i_cols[d, pl.ds(g * _L, _L)]
                acc = acc + u * v
            return acc

        acc = lax.fori_loop(0, _D // 8, dstep,
                            jnp.zeros((_L,), jnp.float32))
        out_v[pl.ds(g * _L, _L)] = acc
        return carry

    lax.fori_loop(0, _BPW // _L, group, 0)

    pltpu.sync_copy(out_v, out_hbm.at[pl.ds(base, _BPW)])


@jax.jit
def kernel(users, items, user_table, item_table):
    mesh = plsc.VectorSubcoreMesh(core_axis_name="c", subcore_axis_name="s")
    k = pl.kernel(
        _body,
        out_type=jax.ShapeDtypeStruct((_B,), jnp.float32),
        mesh=mesh,
        scratch_types=[
            pltpu.VMEM((_NCHUNK, _CHUNK), jnp.int32),
            pltpu.VMEM((_NCHUNK, _CHUNK), jnp.int32),
            pltpu.VMEM((_D, _BPW), jnp.float32),
            pltpu.VMEM((_D, _BPW), jnp.float32),
            pltpu.VMEM((_BPW,), jnp.float32),
            pltpu.SemaphoreType.DMA,
            pltpu.SemaphoreType.DMA,
        ],
        compiler_params=pltpu.CompilerParams(
            needs_layout_passes=False, use_tc_tiling_on_sc=False),
    )
    return k(users, items, user_table.T, item_table.T)


# speedup vs baseline: 10.6936x; 10.6936x over previous
import jax
import jax.numpy as jnp
from jax import lax
from jax.experimental import pallas as pl
from jax.experimental.pallas import tpu as pltpu
from jax.experimental.pallas import tpu_sc as plsc

_B = 16384
_D = 64
_W = 2 * _D
_NC = 2
_NS = 16
_NW = _NC * _NS
_BPW = _B // _NW
_CHUNK = 128
_NCHUNK = _BPW // _CHUNK
_L = 16


def _body(users_hbm, items_hbm, cat_hbm, out_hbm,
          idx_u, idx_i, u_rows, i_rows, out_v, sem_u, sem_i):
    wid = lax.axis_index("s") * _NC + lax.axis_index("c")
    base = wid * _BPW

    for c in range(_NCHUNK):
        off = base + c * _CHUNK
        pltpu.sync_copy(users_hbm.at[pl.ds(off, _CHUNK)], idx_u.at[c])
        pltpu.sync_copy(items_hbm.at[pl.ds(off, _CHUNK)], idx_i.at[c])
        cu = pltpu.async_copy(cat_hbm.at[idx_u.at[c]], u_rows, sem_u)
        ci = pltpu.async_copy(cat_hbm.at[idx_i.at[c]], i_rows, sem_i)
        cu.wait()
        ci.wait()

        def group(g, carry):
            pidx = lax.iota(jnp.int32, _L) + g * _L

            def dstep(d0, acc):
                for dd in range(8):
                    d = d0 * 8 + dd
                    dcol = jnp.zeros((_L,), jnp.int32) + d
                    u = plsc.load_gather(u_rows, [pidx, dcol])
                    v = plsc.load_gather(i_rows, [pidx, dcol + _D])
                    acc = acc + u * v
                return acc

            acc = lax.fori_loop(0, _D // 8, dstep,
                                jnp.zeros((_L,), jnp.float32))
            out_v[pl.ds(c * _CHUNK + g * _L, _L)] = acc
            return carry

        lax.fori_loop(0, _CHUNK // _L, group, 0)

    pltpu.sync_copy(out_v, out_hbm.at[pl.ds(base, _BPW)])


@jax.jit
def kernel(users, items, user_table, item_table):
    cat = jnp.concatenate([user_table, item_table], axis=1)
    mesh = plsc.VectorSubcoreMesh(core_axis_name="c", subcore_axis_name="s")
    k = pl.kernel(
        _body,
        out_type=jax.ShapeDtypeStruct((_B,), jnp.float32),
        mesh=mesh,
        scratch_types=[
            pltpu.VMEM((_NCHUNK, _CHUNK), jnp.int32),
            pltpu.VMEM((_NCHUNK, _CHUNK), jnp.int32),
            pltpu.VMEM((_CHUNK, _W), jnp.float32),
            pltpu.VMEM((_CHUNK, _W), jnp.float32),
            pltpu.VMEM((_BPW,), jnp.float32),
            pltpu.SemaphoreType.DMA,
            pltpu.SemaphoreType.DMA,
        ],
        compiler_params=pltpu.CompilerParams(
            needs_layout_passes=False, use_tc_tiling_on_sc=True),
    )
    return k(users, items, cat)
